# Initial kernel scaffold; baseline (speedup 1.0000x reference)
import jax
import jax.numpy as jnp
from jax.experimental import pallas as pl


def kernel(outputs, indices, memory_bank, W, b, neg_idxs):
    raise NotImplementedError("write your pallas kernel here")


# speedup vs baseline: 1.9635x; 1.9635x over previous
import functools

import jax
import jax.numpy as jnp
from jax import lax
from jax.experimental import pallas as pl
from jax.experimental.pallas import tpu as pltpu
from jax.experimental.pallas import tpu_sc as plsc

B = 1024
M = 1024
EMB = 128
D_OUT = 2048
TAU = 0.07
GAMMA = 0.5

NC = 2
NS = 16
NW = NC * NS
B_PER_W = B // NW
CH = 128
CHUNKS = M // CH
STEPS = B_PER_W * CHUNKS


def _emb_body(x_ref, w_ref, b_ref, o_ref):
    e = jnp.dot(x_ref[...], w_ref[...], preferred_element_type=jnp.float32)
    e = e + b_ref[...]
    n = jnp.sqrt(jnp.sum(e * e, axis=1, keepdims=True))
    o_ref[...] = e / jnp.maximum(n, 1e-12)


def _emb_call(outputs, W, b2d):
    grid = 4
    rows = B // grid
    return pl.pallas_call(
        _emb_body,
        grid=(grid,),
        in_specs=[
            pl.BlockSpec((rows, D_OUT), lambda i: (i, 0)),
            pl.BlockSpec((D_OUT, EMB), lambda i: (0, 0)),
            pl.BlockSpec((1, EMB), lambda i: (0, 0)),
        ],
        out_specs=pl.BlockSpec((rows, EMB), lambda i: (i, 0)),
        out_shape=jax.ShapeDtypeStruct((B, EMB), jnp.float32),
    )(outputs, W, b2d)


def _sc_body(bank, emb, posidx, negidx, neg_out, pos_out,
             idx0, idx1, row0, row1, embv, outv, partials, pidxv, prowv,
             rsem0, rsem1, isem0, isem1, psem):
    wid = lax.axis_index("s") * NC + lax.axis_index("c")
    base = wid * STEPS
    bbase = wid * B_PER_W

    pltpu.sync_copy(emb.at[pl.ds(bbase, B_PER_W)], embv)

    pltpu.sync_copy(negidx.at[base], idx0)
    pltpu.make_async_copy(bank.at[idx0], row0, rsem0).start()
    pltpu.make_async_copy(negidx.at[base + 1], idx1, isem1).start()

    def step(s, idx_cur, idx_oth, row_cur, row_oth,
             rsem_cur, rsem_oth, isem_cur, isem_oth):
        pltpu.make_async_copy(bank.at[idx_cur], row_cur, rsem_cur).wait()

        @pl.when(s + 2 < STEPS)
        def _():
            pltpu.make_async_copy(negidx.at[base + s + 2], idx_cur,
                                  isem_cur).start()

        @pl.when(s + 1 < STEPS)
        def _():
            pltpu.make_async_copy(negidx.at[base + s + 1], idx_oth,
                                  isem_oth).wait()
            pltpu.make_async_copy(bank.at[idx_oth], row_oth, rsem_oth).start()

        bl = s // CHUNKS
        c = s % CHUNKS
        emb_vecs = [embv[bl, pl.ds(16 * k, 16)] for k in range(8)]
        lane = lax.iota(jnp.int32, 16)

        def group(g, carry):
            for i in range(16):
                m = g * 16 + i
                p = [row_cur[m, pl.ds(16 * k, 16)] * emb_vecs[k]
                     for k in range(8)]
                v = ((p[0] + p[1]) + (p[2] + p[3])) + \
                    ((p[4] + p[5]) + (p[6] + p[7]))
                partials[pl.ds(i * 17, 16)] = v
            lane17 = lane * 17
            cols = [plsc.load_gather(partials, [lane17 + l])
                    for l in range(16)]
            t0 = ((cols[0] + cols[1]) + (cols[2] + cols[3])) + \
                 ((cols[4] + cols[5]) + (cols[6] + cols[7]))
            t1 = ((cols[8] + cols[9]) + (cols[10] + cols[11])) + \
                 ((cols[12] + cols[13]) + (cols[14] + cols[15]))
            outv[pl.ds(c * CH + g * 16, 16)] = t0 + t1
            return carry

        lax.fori_loop(0, CH // 16, group, 0)

        @pl.when(c == CHUNKS - 1)
        def _():
            pltpu.sync_copy(outv, neg_out.at[bbase + bl])

    def body(ss, carry):
        step(2 * ss, idx0, idx1, row0, row1, rsem0, rsem1, isem0, isem1)
        step(2 * ss + 1, idx1, idx0, row1, row0, rsem1, rsem0, isem1, isem0)
        return carry

    lax.fori_loop(0, STEPS // 2, body, 0)

    pltpu.sync_copy(posidx.at[pl.ds(bbase, B_PER_W)], pidxv)
    pltpu.make_async_copy(bank.at[pidxv], prowv, psem).start()
    pltpu.make_async_copy(bank.at[pidxv], prowv, psem).wait()
    pltpu.sync_copy(prowv, pos_out.at[pl.ds(bbase, B_PER_W)])


_sc_call = functools.partial(
    pl.kernel,
    out_type=[
        jax.ShapeDtypeStruct((B, M), jnp.float32),
        jax.ShapeDtypeStruct((B, EMB), jnp.float32),
    ],
    mesh=plsc.VectorSubcoreMesh(core_axis_name="c", subcore_axis_name="s",
                                num_cores=NC, num_subcores=NS),
    compiler_params=pltpu.CompilerParams(needs_layout_passes=False),
    scratch_types=[
        pltpu.VMEM((CH,), jnp.int32),
        pltpu.VMEM((CH,), jnp.int32),
        pltpu.VMEM((CH, EMB), jnp.float32),
        pltpu.VMEM((CH, EMB), jnp.float32),
        pltpu.VMEM((B_PER_W, EMB), jnp.float32),
        pltpu.VMEM((M,), jnp.float32),
        pltpu.VMEM((16 * 17,), jnp.float32),
        pltpu.VMEM((B_PER_W,), jnp.int32),
        pltpu.VMEM((B_PER_W, EMB), jnp.float32),
        pltpu.SemaphoreType.DMA,
        pltpu.SemaphoreType.DMA,
        pltpu.SemaphoreType.DMA,
        pltpu.SemaphoreType.DMA,
        pltpu.SemaphoreType.DMA,
    ],
)(_sc_body)


def _loss_body(neg_ref, pm_ref, emb_ref, loss_ref, upd_ref, dl_ref, nl_ref):
    emb = emb_ref[...]
    pm = pm_ref[...]
    u_pos = jnp.sum(emb * pm, axis=1) / TAU
    u_neg = neg_ref[...] * (1.0 / TAU)

    mx = jnp.max(u_neg, axis=1)
    log_C = mx + jnp.log(jnp.sum(jnp.exp(u_neg - mx[:, None]), axis=1))

    mxd = jnp.maximum(u_pos, log_C)
    ldd = mxd + jnp.log(jnp.exp(u_pos - mxd) + jnp.exp(log_C - mxd))
    data_loss = -jnp.sum(u_pos - ldd) / B

    lC = log_C[:, None]
    mxn = jnp.maximum(u_neg, lC)
    lnd = mxn + jnp.log(jnp.exp(u_neg - mxn) + jnp.exp(lC - mxn))
    noise_loss = -jnp.sum(lC - lnd) / B

    loss_ref[...] = jnp.reshape(data_loss + noise_loss, (1, 1))
    dl_ref[...] = jnp.reshape(data_loss, (1, 1))
    nl_ref[...] = jnp.reshape(noise_loss, (1, 1))

    upd = GAMMA * pm + (1.0 - GAMMA) * emb
    n = jnp.sqrt(jnp.sum(upd * upd, axis=1, keepdims=True))
    upd_ref[...] = upd / jnp.maximum(n, 1e-12)


def _loss_call(neg_inner, pos_mem, emb):
    return pl.pallas_call(
        _loss_body,
        out_shape=[
            jax.ShapeDtypeStruct((1, 1), jnp.float32),
            jax.ShapeDtypeStruct((B, EMB), jnp.float32),
            jax.ShapeDtypeStruct((1, 1), jnp.float32),
            jax.ShapeDtypeStruct((1, 1), jnp.float32),
        ],
    )(neg_inner, pos_mem, emb)


def kernel(outputs, indices, memory_bank, W, b, neg_idxs):
    emb = _emb_call(outputs.astype(jnp.float32), W, b.reshape(1, EMB))
    neg_flat = neg_idxs.astype(jnp.int32).reshape(B * CHUNKS, CH)
    neg_inner, pos_mem = _sc_call(memory_bank, emb,
                                  indices.astype(jnp.int32), neg_flat)
    loss, upd, dl, nl = _loss_call(neg_inner, pos_mem, emb)
    return loss[0, 0], upd, dl[0, 0], nl[0, 0]
